# baseline (speedup 1.0000x reference)
import functools

import jax
import jax.numpy as jnp
from jax import lax
from jax.experimental import pallas as pl
from jax.experimental.pallas import tpu as pltpu

ALPHA = 1.0
NN_SIZE = 5
RADIUS = 0.07
H = 0.03
EPS = 1e-12

B = 8
N = 2048
ROWS = 256
NI = N // ROWS
BIG = 3e38


def _loss_kernel(a_ref, p_ref, btg_ref, btp_ref, cd_ref, rep_ref,
                 colmin_ref, bias_ref, acc_ref):
    b = pl.program_id(0)
    i = pl.program_id(1)

    @pl.when(jnp.logical_and(b == 0, i == 0))
    def _init():
        acc_ref[0] = 0.0
        acc_ref[1] = 0.0
        acc_ref[2] = 0.0
        bias_ref[...] = (
            lax.broadcasted_iota(jnp.int32, (N, 1), 0).astype(jnp.float32)
            * jnp.float32(1e-30))

    a = a_ref[0]
    a2 = jnp.sum(a * a, axis=1, keepdims=True)

    btg = btg_ref[0]
    g2 = jnp.sum(btg * btg, axis=0, keepdims=True)
    ab = jnp.dot(a, btg, preferred_element_type=jnp.float32)
    dg = jnp.maximum(a2 + g2 - 2.0 * ab, 0.0)

    rowmin_sum = jnp.sum(jnp.min(dg, axis=1))
    cm = jnp.min(dg, axis=0, keepdims=True)

    @pl.when(i == 0)
    def _cm_init():
        colmin_ref[...] = cm

    @pl.when(i != 0)
    def _cm_acc():
        colmin_ref[...] = jnp.minimum(colmin_ref[...], cm)

    acc_ref[0] = acc_ref[0] + rowmin_sum

    @pl.when(i == NI - 1)
    def _cm_fold():
        acc_ref[1] = acc_ref[1] + jnp.sum(colmin_ref[...])

    p = p_ref[0]
    p2 = jnp.sum(p * p, axis=1, keepdims=True)
    aT = btp_ref[0, :, pl.ds(i * ROWS, ROWS)]
    a2T = jnp.sum(aT * aT, axis=0, keepdims=True)
    apT = jnp.dot(p, aT, preferred_element_type=jnp.float32)
    dpnT = jnp.maximum(p2 + a2T - 2.0 * apT, 0.0)

    dx = p[:, 0:1] - aT[0:1, :]
    dy = p[:, 1:2] - aT[1:2, :]
    dz = p[:, 2:3] - aT[2:3, :]
    exact = dx * dx + dy * dy + dz * dz

    inv_h2 = jnp.float32(1.0 / (H * H))
    rep = jnp.zeros((1, ROWS), jnp.float32)
    vals = dpnT + bias_ref[...]
    for k in range(NN_SIZE):
        m = jnp.min(vals, axis=0, keepdims=True)
        eqm = vals == m
        if k > 0:
            ex = jnp.sum(jnp.where(eqm, exact, 0.0), axis=0, keepdims=True)
            v = jnp.maximum(ex, jnp.float32(EPS))
            dist = jnp.sqrt(v)
            w = jnp.exp(-v * inv_h2)
            rep = rep + (jnp.float32(RADIUS) - dist) * w
        if k < NN_SIZE - 1:
            vals = jnp.where(eqm, BIG, vals)
    acc_ref[2] = acc_ref[2] + jnp.sum(rep)

    @pl.when(jnp.logical_and(b == B - 1, i == NI - 1))
    def _final():
        inv_bn = jnp.float32(1.0 / (B * N))
        cd = (acc_ref[0] + acc_ref[1]) * inv_bn
        cd_ref[...] = jnp.reshape(cd * 100.0, (1, 1))
        rep_ref[...] = jnp.reshape(
            acc_ref[2] * jnp.float32(1.0 / (B * N * (NN_SIZE - 1))), (1, 1))


@functools.partial(jax.jit, static_argnames=())
def _run(pred, gt):
    zeros5 = jnp.zeros((B, N, 5), jnp.float32)
    a_pad = jnp.concatenate([pred, zeros5], axis=2)
    gt_pad = jnp.concatenate([gt, zeros5], axis=2)
    btg = jnp.transpose(gt_pad, (0, 2, 1))
    btp = jnp.transpose(a_pad, (0, 2, 1))

    cd, rep = pl.pallas_call(
        _loss_kernel,
        grid=(B, NI),
        in_specs=[
            pl.BlockSpec((1, ROWS, 8), lambda b, i: (b, i, 0)),
            pl.BlockSpec((1, N, 8), lambda b, i: (b, 0, 0)),
            pl.BlockSpec((1, 8, N), lambda b, i: (b, 0, 0)),
            pl.BlockSpec((1, 8, N), lambda b, i: (b, 0, 0)),
        ],
        out_specs=[
            pl.BlockSpec((1, 1), lambda b, i: (0, 0)),
            pl.BlockSpec((1, 1), lambda b, i: (0, 0)),
        ],
        out_shape=[
            jax.ShapeDtypeStruct((1, 1), jnp.float32),
            jax.ShapeDtypeStruct((1, 1), jnp.float32),
        ],
        scratch_shapes=[
            pltpu.VMEM((1, N), jnp.float32),
            pltpu.VMEM((N, 1), jnp.float32),
            pltpu.SMEM((3,), jnp.float32),
        ],
    )(a_pad, a_pad, btg, btp)
    return cd[0, 0], rep[0, 0]


def kernel(pred, gt, pcd_radius):
    cd, rep = _run(pred, gt)
    return cd, ALPHA * rep


# speedup vs baseline: 1.0878x; 1.0878x over previous
import functools

import jax
import jax.numpy as jnp
from jax import lax
from jax.experimental import pallas as pl
from jax.experimental.pallas import tpu as pltpu

ALPHA = 1.0
NN_SIZE = 5
RADIUS = 0.07
H = 0.03
EPS = 1e-12

B = 8
N = 2048
ROWS = 256
NI = N // ROWS
BIG = 3e38


def _loss_kernel(a_ref, btg_ref, btp_ref, cd_ref, rep_ref, colmin_ref,
                 bias_ref, dm_ref, acc_ref):
    b = pl.program_id(0)
    i = pl.program_id(1)

    @pl.when(jnp.logical_and(b == 0, i == 0))
    def _init():
        acc_ref[0] = 0.0
        acc_ref[1] = 0.0
        acc_ref[2] = 0.0
        bias_ref[...] = (
            lax.broadcasted_iota(jnp.int32, (1, N), 1).astype(jnp.float32)
            * jnp.float32(1e-30))
        ci = lax.broadcasted_iota(jnp.int32, (ROWS, N), 1)
        ri = lax.broadcasted_iota(jnp.int32, (ROWS, N), 0)
        dm_ref[...] = (ci - ri).astype(jnp.float32)

    a = a_ref[0]
    a2 = jnp.sum(a * a, axis=1, keepdims=True)

    btg = btg_ref[0]
    g2 = jnp.sum(btg * btg, axis=0, keepdims=True)
    ab = jnp.dot(a, btg, preferred_element_type=jnp.float32)
    dg = jnp.maximum(a2 + g2 - 2.0 * ab, 0.0)

    rowmin_sum = jnp.sum(jnp.min(dg, axis=1))
    cm = jnp.min(dg, axis=0, keepdims=True)

    @pl.when(i == 0)
    def _cm_init():
        colmin_ref[...] = cm

    @pl.when(i != 0)
    def _cm_acc():
        colmin_ref[...] = jnp.minimum(colmin_ref[...], cm)

    acc_ref[0] = acc_ref[0] + rowmin_sum

    @pl.when(i == NI - 1)
    def _cm_fold():
        acc_ref[1] = acc_ref[1] + jnp.sum(colmin_ref[...])

    btp = btp_ref[0]
    p2 = jnp.sum(btp * btp, axis=0, keepdims=True)
    ap = jnp.dot(a, btp, preferred_element_type=jnp.float32)
    dpn = jnp.maximum(a2 + p2 - 2.0 * ap, 0.0)

    ap_hi = jnp.dot(a, btp, preferred_element_type=jnp.float32,
                    precision=jax.lax.Precision.HIGHEST)
    exact = jnp.where(dm_ref[...] == jnp.float32(ROWS) * i.astype(jnp.float32),
                      0.0, a2 + p2 - 2.0 * ap_hi)

    inv_h2 = jnp.float32(1.0 / (H * H))
    rep = jnp.zeros((ROWS, 1), jnp.float32)
    vals = dpn + bias_ref[...]
    for k in range(NN_SIZE):
        m = jnp.min(vals, axis=1, keepdims=True)
        eqm = vals == m
        if k > 0:
            ex = jnp.sum(jnp.where(eqm, exact, 0.0), axis=1, keepdims=True)
            v = jnp.maximum(ex, jnp.float32(EPS))
            dist = jnp.sqrt(v)
            w = jnp.exp(-v * inv_h2)
            rep = rep + (jnp.float32(RADIUS) - dist) * w
        if k < NN_SIZE - 1:
            vals = jnp.where(eqm, BIG, vals)
    acc_ref[2] = acc_ref[2] + jnp.sum(rep)

    @pl.when(jnp.logical_and(b == B - 1, i == NI - 1))
    def _final():
        inv_bn = jnp.float32(1.0 / (B * N))
        cd = (acc_ref[0] + acc_ref[1]) * inv_bn
        cd_ref[...] = jnp.reshape(cd * 100.0, (1, 1))
        rep_ref[...] = jnp.reshape(
            acc_ref[2] * jnp.float32(1.0 / (B * N * (NN_SIZE - 1))), (1, 1))


@functools.partial(jax.jit, static_argnames=())
def _run(pred, gt):
    zeros5 = jnp.zeros((B, N, 5), jnp.float32)
    a_pad = jnp.concatenate([pred, zeros5], axis=2)
    gt_pad = jnp.concatenate([gt, zeros5], axis=2)
    btg = jnp.transpose(gt_pad, (0, 2, 1))
    btp = jnp.transpose(a_pad, (0, 2, 1))

    cd, rep = pl.pallas_call(
        _loss_kernel,
        grid=(B, NI),
        in_specs=[
            pl.BlockSpec((1, ROWS, 8), lambda b, i: (b, i, 0)),
            pl.BlockSpec((1, 8, N), lambda b, i: (b, 0, 0)),
            pl.BlockSpec((1, 8, N), lambda b, i: (b, 0, 0)),
        ],
        out_specs=[
            pl.BlockSpec((1, 1), lambda b, i: (0, 0)),
            pl.BlockSpec((1, 1), lambda b, i: (0, 0)),
        ],
        out_shape=[
            jax.ShapeDtypeStruct((1, 1), jnp.float32),
            jax.ShapeDtypeStruct((1, 1), jnp.float32),
        ],
        scratch_shapes=[
            pltpu.VMEM((1, N), jnp.float32),
            pltpu.VMEM((1, N), jnp.float32),
            pltpu.VMEM((ROWS, N), jnp.float32),
            pltpu.SMEM((3,), jnp.float32),
        ],
    )(a_pad, btg, btp)
    return cd[0, 0], rep[0, 0]


def kernel(pred, gt, pcd_radius):
    cd, rep = _run(pred, gt)
    return cd, ALPHA * rep


# speedup vs baseline: 1.3084x; 1.2028x over previous
import functools

import jax
import jax.numpy as jnp
from jax import lax
from jax.experimental import pallas as pl
from jax.experimental.pallas import tpu as pltpu

ALPHA = 1.0
NN_SIZE = 5
RADIUS = 0.07
H = 0.03
EPS = 1e-12

B = 8
N = 2048
ROWS = 512
NI = N // ROWS
BIG = 3e38


def _loss_kernel(a_ref, btg_ref, btp_ref, cd_ref, rep_ref, colmin_ref,
                 bias_ref, acc_ref):
    b = pl.program_id(0)
    i = pl.program_id(1)

    @pl.when(jnp.logical_and(b == 0, i == 0))
    def _init():
        acc_ref[0] = 0.0
        acc_ref[1] = 0.0
        acc_ref[2] = 0.0
        bias_ref[...] = (
            lax.broadcasted_iota(jnp.int32, (1, N), 1).astype(jnp.float32)
            * jnp.float32(1e-30))

    a = a_ref[0]
    a2 = jnp.sum(a * a, axis=1, keepdims=True)

    btg = btg_ref[0]
    g2 = jnp.sum(btg * btg, axis=0, keepdims=True)
    ab = jnp.dot(a, btg, preferred_element_type=jnp.float32)
    dg = jnp.maximum(a2 + g2 - 2.0 * ab, 0.0)

    rowmin_sum = jnp.sum(jnp.min(dg, axis=1))
    cm = jnp.min(dg, axis=0, keepdims=True)

    @pl.when(i == 0)
    def _cm_init():
        colmin_ref[...] = cm

    @pl.when(i != 0)
    def _cm_acc():
        colmin_ref[...] = jnp.minimum(colmin_ref[...], cm)

    acc_ref[0] = acc_ref[0] + rowmin_sum

    @pl.when(i == NI - 1)
    def _cm_fold():
        acc_ref[1] = acc_ref[1] + jnp.sum(colmin_ref[...])

    btp = btp_ref[0]
    p2 = jnp.sum(btp * btp, axis=0, keepdims=True)
    ap = jnp.dot(a, btp, preferred_element_type=jnp.float32)
    dpn = jnp.maximum(a2 + p2 - 2.0 * ap, 0.0)

    dx = a[:, 0:1] - btp[0:1, :]
    dy = a[:, 1:2] - btp[1:2, :]
    dz = a[:, 2:3] - btp[2:3, :]
    exact = dx * dx + dy * dy + dz * dz

    inv_h2 = jnp.float32(1.0 / (H * H))
    rep = jnp.zeros((ROWS, 1), jnp.float32)
    vals = dpn + bias_ref[...]
    for k in range(NN_SIZE):
        m = jnp.min(vals, axis=1, keepdims=True)
        eqm = vals == m
        if k > 0:
            ex = jnp.sum(jnp.where(eqm, exact, 0.0), axis=1, keepdims=True)
            v = jnp.maximum(ex, jnp.float32(EPS))
            dist = jnp.sqrt(v)
            w = jnp.exp(-v * inv_h2)
            rep = rep + (jnp.float32(RADIUS) - dist) * w
        if k < NN_SIZE - 1:
            vals = jnp.where(eqm, BIG, vals)
    acc_ref[2] = acc_ref[2] + jnp.sum(rep)

    @pl.when(jnp.logical_and(b == B - 1, i == NI - 1))
    def _final():
        inv_bn = jnp.float32(1.0 / (B * N))
        cd = (acc_ref[0] + acc_ref[1]) * inv_bn
        cd_ref[...] = jnp.reshape(cd * 100.0, (1, 1))
        rep_ref[...] = jnp.reshape(
            acc_ref[2] * jnp.float32(1.0 / (B * N * (NN_SIZE - 1))), (1, 1))


@functools.partial(jax.jit, static_argnames=())
def _run(pred, gt):
    zeros5 = jnp.zeros((B, N, 5), jnp.float32)
    a_pad = jnp.concatenate([pred, zeros5], axis=2)
    gt_pad = jnp.concatenate([gt, zeros5], axis=2)
    btg = jnp.transpose(gt_pad, (0, 2, 1))
    btp = jnp.transpose(a_pad, (0, 2, 1))

    cd, rep = pl.pallas_call(
        _loss_kernel,
        grid=(B, NI),
        in_specs=[
            pl.BlockSpec((1, ROWS, 8), lambda b, i: (b, i, 0)),
            pl.BlockSpec((1, 8, N), lambda b, i: (b, 0, 0)),
            pl.BlockSpec((1, 8, N), lambda b, i: (b, 0, 0)),
        ],
        out_specs=[
            pl.BlockSpec((1, 1), lambda b, i: (0, 0)),
            pl.BlockSpec((1, 1), lambda b, i: (0, 0)),
        ],
        out_shape=[
            jax.ShapeDtypeStruct((1, 1), jnp.float32),
            jax.ShapeDtypeStruct((1, 1), jnp.float32),
        ],
        scratch_shapes=[
            pltpu.VMEM((1, N), jnp.float32),
            pltpu.VMEM((1, N), jnp.float32),
            pltpu.SMEM((3,), jnp.float32),
        ],
    )(a_pad, btg, btp)
    return cd[0, 0], rep[0, 0]


def kernel(pred, gt, pcd_radius):
    cd, rep = _run(pred, gt)
    return cd, ALPHA * rep


# speedup vs baseline: 1.3621x; 1.0410x over previous
import functools

import jax
import jax.numpy as jnp
from jax import lax
from jax.experimental import pallas as pl
from jax.experimental.pallas import tpu as pltpu

ALPHA = 1.0
NN_SIZE = 5
RADIUS = 0.07
H = 0.03
EPS = 1e-12

B = 8
N = 2048
ROWS = 1024
NI = N // ROWS
BIG = 3e38


def _loss_kernel(a_ref, btg_ref, btp_ref, cd_ref, rep_ref, colmin_ref,
                 bias_ref, acc_ref):
    b = pl.program_id(0)
    i = pl.program_id(1)

    @pl.when(jnp.logical_and(b == 0, i == 0))
    def _init():
        acc_ref[0] = 0.0
        acc_ref[1] = 0.0
        acc_ref[2] = 0.0
        bias_ref[...] = (
            lax.broadcasted_iota(jnp.int32, (1, N), 1).astype(jnp.float32)
            * jnp.float32(1e-30))

    a = a_ref[0]
    a2 = jnp.sum(a * a, axis=1, keepdims=True)

    btg = btg_ref[0]
    g2 = jnp.sum(btg * btg, axis=0, keepdims=True)
    ab = jnp.dot(a, btg, preferred_element_type=jnp.float32)
    dg = jnp.maximum(a2 + g2 - 2.0 * ab, 0.0)

    rowmin_sum = jnp.sum(jnp.min(dg, axis=1))
    cm = jnp.min(dg, axis=0, keepdims=True)

    @pl.when(i == 0)
    def _cm_init():
        colmin_ref[...] = cm

    @pl.when(i != 0)
    def _cm_acc():
        colmin_ref[...] = jnp.minimum(colmin_ref[...], cm)

    acc_ref[0] = acc_ref[0] + rowmin_sum

    @pl.when(i == NI - 1)
    def _cm_fold():
        acc_ref[1] = acc_ref[1] + jnp.sum(colmin_ref[...])

    btp = btp_ref[0]
    p2 = jnp.sum(btp * btp, axis=0, keepdims=True)
    ap = jnp.dot(a, btp, preferred_element_type=jnp.float32)
    dpn = jnp.maximum(a2 + p2 - 2.0 * ap, 0.0)

    dx = a[:, 0:1] - btp[0:1, :]
    dy = a[:, 1:2] - btp[1:2, :]
    dz = a[:, 2:3] - btp[2:3, :]
    exact = dx * dx + dy * dy + dz * dz

    inv_h2 = jnp.float32(1.0 / (H * H))
    rep = jnp.zeros((ROWS, 1), jnp.float32)
    vals = dpn + bias_ref[...]
    for k in range(NN_SIZE):
        m = jnp.min(vals, axis=1, keepdims=True)
        eqm = vals == m
        if k > 0:
            ex = jnp.sum(jnp.where(eqm, exact, 0.0), axis=1, keepdims=True)
            v = jnp.maximum(ex, jnp.float32(EPS))
            dist = jnp.sqrt(v)
            w = jnp.exp(-v * inv_h2)
            rep = rep + (jnp.float32(RADIUS) - dist) * w
        if k < NN_SIZE - 1:
            vals = jnp.where(eqm, BIG, vals)
    acc_ref[2] = acc_ref[2] + jnp.sum(rep)

    @pl.when(jnp.logical_and(b == B - 1, i == NI - 1))
    def _final():
        inv_bn = jnp.float32(1.0 / (B * N))
        cd = (acc_ref[0] + acc_ref[1]) * inv_bn
        cd_ref[...] = jnp.reshape(cd * 100.0, (1, 1))
        rep_ref[...] = jnp.reshape(
            acc_ref[2] * jnp.float32(1.0 / (B * N * (NN_SIZE - 1))), (1, 1))


@functools.partial(jax.jit, static_argnames=())
def _run(pred, gt):
    zeros5 = jnp.zeros((B, N, 5), jnp.float32)
    a_pad = jnp.concatenate([pred, zeros5], axis=2)
    gt_pad = jnp.concatenate([gt, zeros5], axis=2)
    btg = jnp.transpose(gt_pad, (0, 2, 1))
    btp = jnp.transpose(a_pad, (0, 2, 1))

    cd, rep = pl.pallas_call(
        _loss_kernel,
        grid=(B, NI),
        in_specs=[
            pl.BlockSpec((1, ROWS, 8), lambda b, i: (b, i, 0)),
            pl.BlockSpec((1, 8, N), lambda b, i: (b, 0, 0)),
            pl.BlockSpec((1, 8, N), lambda b, i: (b, 0, 0)),
        ],
        out_specs=[
            pl.BlockSpec((1, 1), lambda b, i: (0, 0)),
            pl.BlockSpec((1, 1), lambda b, i: (0, 0)),
        ],
        out_shape=[
            jax.ShapeDtypeStruct((1, 1), jnp.float32),
            jax.ShapeDtypeStruct((1, 1), jnp.float32),
        ],
        scratch_shapes=[
            pltpu.VMEM((1, N), jnp.float32),
            pltpu.VMEM((1, N), jnp.float32),
            pltpu.SMEM((3,), jnp.float32),
        ],
    )(a_pad, btg, btp)
    return cd[0, 0], rep[0, 0]


def kernel(pred, gt, pcd_radius):
    cd, rep = _run(pred, gt)
    return cd, ALPHA * rep


# speedup vs baseline: 1.4286x; 1.0488x over previous
import functools

import jax
import jax.numpy as jnp
from jax import lax
from jax.experimental import pallas as pl
from jax.experimental.pallas import tpu as pltpu

ALPHA = 1.0
NN_SIZE = 5
RADIUS = 0.07
H = 0.03
EPS = 1e-12

B = 8
N = 2048
ROWS = 2048
NI = N // ROWS
BIG = 3e38


def _loss_kernel(a_ref, btg_ref, btp_ref, cd_ref, rep_ref, colmin_ref,
                 bias_ref, acc_ref):
    b = pl.program_id(0)
    i = pl.program_id(1)

    @pl.when(jnp.logical_and(b == 0, i == 0))
    def _init():
        acc_ref[0] = 0.0
        acc_ref[1] = 0.0
        acc_ref[2] = 0.0
        bias_ref[...] = (
            lax.broadcasted_iota(jnp.int32, (1, N), 1).astype(jnp.float32)
            * jnp.float32(1e-30))

    a = a_ref[0]
    a2 = jnp.sum(a * a, axis=1, keepdims=True)

    btg = btg_ref[0]
    g2 = jnp.sum(btg * btg, axis=0, keepdims=True)
    ab = jnp.dot(a, btg, preferred_element_type=jnp.float32)
    dg = jnp.maximum(a2 + g2 - 2.0 * ab, 0.0)

    rowmin_sum = jnp.sum(jnp.min(dg, axis=1))
    cm = jnp.min(dg, axis=0, keepdims=True)

    @pl.when(i == 0)
    def _cm_init():
        colmin_ref[...] = cm

    @pl.when(i != 0)
    def _cm_acc():
        colmin_ref[...] = jnp.minimum(colmin_ref[...], cm)

    acc_ref[0] = acc_ref[0] + rowmin_sum

    @pl.when(i == NI - 1)
    def _cm_fold():
        acc_ref[1] = acc_ref[1] + jnp.sum(colmin_ref[...])

    btp = btp_ref[0]
    p2 = jnp.sum(btp * btp, axis=0, keepdims=True)
    ap = jnp.dot(a, btp, preferred_element_type=jnp.float32)
    dpn = jnp.maximum(a2 + p2 - 2.0 * ap, 0.0)

    dx = a[:, 0:1] - btp[0:1, :]
    dy = a[:, 1:2] - btp[1:2, :]
    dz = a[:, 2:3] - btp[2:3, :]
    exact = dx * dx + dy * dy + dz * dz

    inv_h2 = jnp.float32(1.0 / (H * H))
    rep = jnp.zeros((ROWS, 1), jnp.float32)
    vals = dpn + bias_ref[...]
    for k in range(NN_SIZE):
        m = jnp.min(vals, axis=1, keepdims=True)
        eqm = vals == m
        if k > 0:
            ex = jnp.sum(jnp.where(eqm, exact, 0.0), axis=1, keepdims=True)
            v = jnp.maximum(ex, jnp.float32(EPS))
            dist = jnp.sqrt(v)
            w = jnp.exp(-v * inv_h2)
            rep = rep + (jnp.float32(RADIUS) - dist) * w
        if k < NN_SIZE - 1:
            vals = jnp.where(eqm, BIG, vals)
    acc_ref[2] = acc_ref[2] + jnp.sum(rep)

    @pl.when(jnp.logical_and(b == B - 1, i == NI - 1))
    def _final():
        inv_bn = jnp.float32(1.0 / (B * N))
        cd = (acc_ref[0] + acc_ref[1]) * inv_bn
        cd_ref[...] = jnp.reshape(cd * 100.0, (1, 1))
        rep_ref[...] = jnp.reshape(
            acc_ref[2] * jnp.float32(1.0 / (B * N * (NN_SIZE - 1))), (1, 1))


@functools.partial(jax.jit, static_argnames=())
def _run(pred, gt):
    zeros5 = jnp.zeros((B, N, 5), jnp.float32)
    a_pad = jnp.concatenate([pred, zeros5], axis=2)
    gt_pad = jnp.concatenate([gt, zeros5], axis=2)
    btg = jnp.transpose(gt_pad, (0, 2, 1))
    btp = jnp.transpose(a_pad, (0, 2, 1))

    cd, rep = pl.pallas_call(
        _loss_kernel,
        grid=(B, NI),
        in_specs=[
            pl.BlockSpec((1, ROWS, 8), lambda b, i: (b, i, 0)),
            pl.BlockSpec((1, 8, N), lambda b, i: (b, 0, 0)),
            pl.BlockSpec((1, 8, N), lambda b, i: (b, 0, 0)),
        ],
        out_specs=[
            pl.BlockSpec((1, 1), lambda b, i: (0, 0)),
            pl.BlockSpec((1, 1), lambda b, i: (0, 0)),
        ],
        out_shape=[
            jax.ShapeDtypeStruct((1, 1), jnp.float32),
            jax.ShapeDtypeStruct((1, 1), jnp.float32),
        ],
        scratch_shapes=[
            pltpu.VMEM((1, N), jnp.float32),
            pltpu.VMEM((1, N), jnp.float32),
            pltpu.SMEM((3,), jnp.float32),
        ],
    )(a_pad, btg, btp)
    return cd[0, 0], rep[0, 0]


def kernel(pred, gt, pcd_radius):
    cd, rep = _run(pred, gt)
    return cd, ALPHA * rep


# speedup vs baseline: 1.4309x; 1.0016x over previous
import functools

import jax
import jax.numpy as jnp
from jax import lax
from jax.experimental import pallas as pl
from jax.experimental.pallas import tpu as pltpu

ALPHA = 1.0
NN_SIZE = 5
RADIUS = 0.07
H = 0.03
EPS = 1e-12

B = 8
N = 2048
BIG = 3e38


def _loss_kernel(a_ref, btg_ref, btp_ref, cd_ref, rep_ref, bias_ref, acc_ref):
    b = pl.program_id(0)

    @pl.when(b == 0)
    def _init():
        acc_ref[0] = 0.0
        acc_ref[1] = 0.0
        acc_ref[2] = 0.0
        bias_ref[...] = (
            lax.broadcasted_iota(jnp.int32, (1, N), 1).astype(jnp.float32)
            * jnp.float32(1e-30))

    a = a_ref[0]
    a2 = jnp.sum(a * a, axis=1, keepdims=True)

    btg = btg_ref[0]
    g2 = jnp.sum(btg * btg, axis=0, keepdims=True)
    ab = jnp.dot(a, btg, preferred_element_type=jnp.float32)
    dg = jnp.maximum(a2 + g2 - 2.0 * ab, 0.0)

    acc_ref[0] = acc_ref[0] + jnp.sum(jnp.min(dg, axis=1))
    acc_ref[1] = acc_ref[1] + jnp.sum(jnp.min(dg, axis=0))

    btp = btp_ref[0]
    p2 = jnp.sum(btp * btp, axis=0, keepdims=True)
    ap = jnp.dot(a, btp, preferred_element_type=jnp.float32)
    vals = jnp.maximum(a2 + p2 - 2.0 * ap, 0.0) + bias_ref[...]

    dx = a[:, 0:1] - btp[0:1, :]
    dy = a[:, 1:2] - btp[1:2, :]
    dz = a[:, 2:3] - btp[2:3, :]
    exact = dx * dx + dy * dy + dz * dz

    inv_h2 = jnp.float32(1.0 / (H * H))
    rep = jnp.zeros((N, 1), jnp.float32)
    for k in range(NN_SIZE):
        m = jnp.min(vals, axis=1, keepdims=True)
        eqm = vals == m
        if k > 0:
            ex = jnp.sum(jnp.where(eqm, exact, 0.0), axis=1, keepdims=True)
            v = jnp.maximum(ex, jnp.float32(EPS))
            dist = jnp.sqrt(v)
            w = jnp.exp(-v * inv_h2)
            rep = rep + (jnp.float32(RADIUS) - dist) * w
        if k < NN_SIZE - 1:
            vals = jnp.where(eqm, BIG, vals)
    acc_ref[2] = acc_ref[2] + jnp.sum(rep)

    @pl.when(b == B - 1)
    def _final():
        inv_bn = jnp.float32(1.0 / (B * N))
        cd = (acc_ref[0] + acc_ref[1]) * inv_bn
        cd_ref[...] = jnp.reshape(cd * 100.0, (1, 1))
        rep_ref[...] = jnp.reshape(
            acc_ref[2] * jnp.float32(1.0 / (B * N * (NN_SIZE - 1))), (1, 1))


@functools.partial(jax.jit, static_argnames=())
def _run(pred, gt):
    zeros5 = jnp.zeros((B, N, 5), jnp.float32)
    a_pad = jnp.concatenate([pred, zeros5], axis=2)
    gt_pad = jnp.concatenate([gt, zeros5], axis=2)
    btg = jnp.transpose(gt_pad, (0, 2, 1))
    btp = jnp.transpose(a_pad, (0, 2, 1))

    cd, rep = pl.pallas_call(
        _loss_kernel,
        grid=(B,),
        in_specs=[
            pl.BlockSpec((1, N, 8), lambda b: (b, 0, 0)),
            pl.BlockSpec((1, 8, N), lambda b: (b, 0, 0)),
            pl.BlockSpec((1, 8, N), lambda b: (b, 0, 0)),
        ],
        out_specs=[
            pl.BlockSpec((1, 1), lambda b: (0, 0)),
            pl.BlockSpec((1, 1), lambda b: (0, 0)),
        ],
        out_shape=[
            jax.ShapeDtypeStruct((1, 1), jnp.float32),
            jax.ShapeDtypeStruct((1, 1), jnp.float32),
        ],
        scratch_shapes=[
            pltpu.VMEM((1, N), jnp.float32),
            pltpu.SMEM((3,), jnp.float32),
        ],
    )(a_pad, btg, btp)
    return cd[0, 0], rep[0, 0]


def kernel(pred, gt, pcd_radius):
    cd, rep = _run(pred, gt)
    return cd, ALPHA * rep


# speedup vs baseline: 1.4497x; 1.0132x over previous
import functools

import jax
import jax.numpy as jnp
from jax import lax
from jax.experimental import pallas as pl
from jax.experimental.pallas import tpu as pltpu

ALPHA = 1.0
NN_SIZE = 5
RADIUS = 0.07
H = 0.03
EPS = 1e-12

B = 8
N = 2048
BIG = 3e38


def _loss_kernel(a_ref, btg_ref, btp_ref, cd_ref, rep_ref, bias_ref, acc_ref):
    b = pl.program_id(0)

    @pl.when(b == 0)
    def _init():
        acc_ref[0] = 0.0
        acc_ref[1] = 0.0
        acc_ref[2] = 0.0
        bias_ref[...] = (
            lax.broadcasted_iota(jnp.int32, (1, N), 1).astype(jnp.float32)
            * jnp.float32(1e-30))

    a = a_ref[0]
    a2 = jnp.sum(a * a, axis=1, keepdims=True)

    btg = btg_ref[0]
    g2 = jnp.sum(btg * btg, axis=0, keepdims=True)
    ab = jnp.dot(a, btg, preferred_element_type=jnp.float32)
    dg = a2 + g2 - 2.0 * ab

    acc_ref[0] = acc_ref[0] + jnp.sum(jnp.maximum(jnp.min(dg, axis=1), 0.0))
    acc_ref[1] = acc_ref[1] + jnp.sum(jnp.maximum(jnp.min(dg, axis=0), 0.0))

    btp = btp_ref[0]
    p2 = jnp.sum(btp * btp, axis=0, keepdims=True)
    ap = jnp.dot(a, btp, preferred_element_type=jnp.float32)
    vals = jnp.maximum(a2 + p2 - 2.0 * ap, 0.0) + bias_ref[...]

    dx = a[:, 0:1] - btp[0:1, :]
    dy = a[:, 1:2] - btp[1:2, :]
    dz = a[:, 2:3] - btp[2:3, :]
    exact = dx * dx + dy * dy + dz * dz

    inv_h2 = jnp.float32(1.0 / (H * H))
    rep = jnp.zeros((N, 1), jnp.float32)
    for k in range(NN_SIZE):
        m = jnp.min(vals, axis=1, keepdims=True)
        eqm = vals == m
        if k > 0:
            ex = jnp.sum(jnp.where(eqm, exact, 0.0), axis=1, keepdims=True)
            v = jnp.maximum(ex, jnp.float32(EPS))
            dist = jnp.sqrt(v)
            w = jnp.exp(-v * inv_h2)
            rep = rep + (jnp.float32(RADIUS) - dist) * w
        if k < NN_SIZE - 1:
            vals = jnp.where(eqm, BIG, vals)
    acc_ref[2] = acc_ref[2] + jnp.sum(rep)

    @pl.when(b == B - 1)
    def _final():
        inv_bn = jnp.float32(1.0 / (B * N))
        cd = (acc_ref[0] + acc_ref[1]) * inv_bn
        cd_ref[...] = jnp.reshape(cd * 100.0, (1, 1))
        rep_ref[...] = jnp.reshape(
            acc_ref[2] * jnp.float32(1.0 / (B * N * (NN_SIZE - 1))), (1, 1))


@functools.partial(jax.jit, static_argnames=())
def _run(pred, gt):
    zeros5 = jnp.zeros((B, N, 5), jnp.float32)
    a_pad = jnp.concatenate([pred, zeros5], axis=2)
    gt_pad = jnp.concatenate([gt, zeros5], axis=2)
    btg = jnp.transpose(gt_pad, (0, 2, 1))
    btp = jnp.transpose(a_pad, (0, 2, 1))

    cd, rep = pl.pallas_call(
        _loss_kernel,
        grid=(B,),
        in_specs=[
            pl.BlockSpec((1, N, 8), lambda b: (b, 0, 0)),
            pl.BlockSpec((1, 8, N), lambda b: (b, 0, 0)),
            pl.BlockSpec((1, 8, N), lambda b: (b, 0, 0)),
        ],
        out_specs=[
            pl.BlockSpec((1, 1), lambda b: (0, 0)),
            pl.BlockSpec((1, 1), lambda b: (0, 0)),
        ],
        out_shape=[
            jax.ShapeDtypeStruct((1, 1), jnp.float32),
            jax.ShapeDtypeStruct((1, 1), jnp.float32),
        ],
        scratch_shapes=[
            pltpu.VMEM((1, N), jnp.float32),
            pltpu.SMEM((3,), jnp.float32),
        ],
    )(a_pad, btg, btp)
    return cd[0, 0], rep[0, 0]


def kernel(pred, gt, pcd_radius):
    cd, rep = _run(pred, gt)
    return cd, ALPHA * rep
